# baseline (speedup 1.0000x reference)
import functools

import jax
import jax.numpy as jnp
from jax import lax
from jax.experimental import pallas as pl
from jax.experimental.pallas import tpu as pltpu
from jax.experimental.pallas import tpu_sc as plsc

N = 10000
D = 128
HD = D // 2
M = 3
E = 320000
P = 200000

NC = 2
NS = 16
NW = NC * NS

ECH = 128
NBUF = 4
FAST_C = 0
ECF = 132
ECS = 28
E_CHUNKS = NS * (ECF + ECS)
E_STAGE = ECF * ECH
E_PAD = (NS * ECF + (NS - 1) * ECS + ECF) * ECH

PCH = 56
PCF = 188
PCS = 36
P_CHUNKS = NS * (PCF + PCS)
P_STAGE = PCF * PCH
P_PAD = (NS * PCF + (NS - 1) * PCS + PCF) * PCH

N_PAD = 10008
NPS = N // NS
ZCH = 125

_mesh = plsc.VectorSubcoreMesh(core_axis_name="c", subcore_axis_name="s")


def _zero_rows(ref, nrows, width):
    def body(i, _):
        for j in range(width // 16):
            ref[i, pl.ds(j * 16, 16)] = jnp.zeros((16,), jnp.float32)
        return jnp.int32(0)
    lax.fori_loop(jnp.int32(0), jnp.int32(nrows), body, jnp.int32(0))


def _ones_rows(ref, nrows):
    def body(i, _):
        ref[i, :] = jnp.ones((16,), jnp.float32)
        return jnp.int32(0)
    lax.fori_loop(jnp.int32(0), jnp.int32(nrows), body, jnp.int32(0))


def _seg_pass(feat_hbm, srcv, dstv, bufs, accsh, semg, sems, nk, lastc,
              deg=None):
    def g_src(chunk):
        return feat_hbm.at[srcv.at[pl.ds(chunk * ECH, ECH)]]

    for j in range(NBUF):
        pltpu.async_copy(g_src(jnp.int32(j)), bufs[j], semg)

    def body(k, _):
        for j in range(NBUF):
            i = k * NBUF + j
            pltpu.make_async_copy(g_src(i), bufs[j], semg).wait()
            pltpu.async_copy(bufs[j], accsh.at[dstv.at[i]], sems, add=True)
            if deg is not None:
                onesv, degsh, semd = deg
                pltpu.async_copy(onesv, degsh.at[dstv.at[i]], semd, add=True)
                pltpu.make_async_copy(onesv, degsh.at[dstv.at[i]], semd).wait()
            pltpu.make_async_copy(bufs[j], accsh.at[dstv.at[i]], sems).wait()
            nxt = jnp.minimum(i + NBUF, lastc)
            pltpu.async_copy(g_src(nxt), bufs[j], semg)
        return jnp.int32(0)

    lax.fori_loop(jnp.int32(0), nk, body, jnp.int32(0))
    for j in range(NBUF):
        pltpu.make_async_copy(g_src(lastc), bufs[j], semg).wait()


@functools.partial(
    pl.kernel,
    out_type=(
        jax.ShapeDtypeStruct((NC, 2, N_PAD, HD), jnp.float32),
        jax.ShapeDtypeStruct((NC, N_PAD, 16), jnp.float32),
    ),
    mesh=_mesh,
    compiler_params=pltpu.CompilerParams(use_tc_tiling_on_sc=False),
    scratch_types=[
        pltpu.VMEM((E_STAGE,), jnp.int32),
        pltpu.VMEM((ECF, ECH), jnp.int32),
        [pltpu.VMEM((ECH, HD), jnp.float32) for _ in range(NBUF)],
        pltpu.VMEM((ECH, 16), jnp.float32),
        pltpu.VMEM((ZCH, HD), jnp.float32),
        pltpu.VMEM((ZCH, 16), jnp.float32),
        pltpu.VMEM_SHARED((N_PAD, HD), jnp.float32),
        pltpu.VMEM_SHARED((N_PAD, 16), jnp.float32),
        pltpu.SemaphoreType.DMA,
        pltpu.SemaphoreType.DMA,
        pltpu.SemaphoreType.DMA,
    ],
)
def _seg1(src_hbm, dst_hbm, x0_hbm, x1_hbm, acc_out, deg_out,
          srcv, dstv, bufs, onesv, zacc, zdeg, accsh, degsh,
          semg, sems, semd):
    c = lax.axis_index("c")
    s = lax.axis_index("s")
    wid = c * NS + s

    _zero_rows(zacc, ZCH, HD)
    _zero_rows(zdeg, ZCH, 16)
    _ones_rows(onesv, ECH)

    base_n = s * NPS
    fast = c == jnp.int32(FAST_C)
    base_ch = jnp.where(fast, s * ECF, NS * ECF + s * ECS)
    nch = jnp.where(fast, jnp.int32(ECF), jnp.int32(ECS))
    nk = jnp.where(fast, jnp.int32(ECF // NBUF), jnp.int32(ECS // NBUF))
    lastc = nch - 1
    pltpu.sync_copy(src_hbm.at[pl.ds(base_ch * ECH, E_STAGE)], srcv)
    pltpu.sync_copy(dst_hbm.at[pl.ds(base_ch, ECF)], dstv)

    for half, xh_hbm in enumerate((x0_hbm, x1_hbm)):
        for q in range(NPS // ZCH):
            pltpu.sync_copy(zacc, accsh.at[pl.ds(base_n + q * ZCH, ZCH)])
            if half == 0:
                pltpu.sync_copy(zdeg, degsh.at[pl.ds(base_n + q * ZCH, ZCH)])
        plsc.subcore_barrier()

        _seg_pass(xh_hbm, srcv, dstv, bufs, accsh, semg, sems, nk, lastc,
                  deg=(onesv, degsh, semd) if half == 0 else None)
        plsc.subcore_barrier()

        pltpu.sync_copy(accsh.at[pl.ds(base_n, NPS)],
                        acc_out.at[c, jnp.int32(half), pl.ds(base_n, NPS)])
        if half == 0:
            pltpu.sync_copy(degsh.at[pl.ds(base_n, NPS)],
                            deg_out.at[c, pl.ds(base_n, NPS)])
        plsc.subcore_barrier()


@functools.partial(
    pl.kernel,
    out_type=jax.ShapeDtypeStruct((NC, M, 2, N_PAD, HD), jnp.float32),
    mesh=_mesh,
    compiler_params=pltpu.CompilerParams(use_tc_tiling_on_sc=False),
    scratch_types=[
        pltpu.VMEM((E_STAGE,), jnp.int32),
        pltpu.VMEM((ECF, ECH), jnp.int32),
        [pltpu.VMEM((ECH, HD), jnp.float32) for _ in range(NBUF)],
        pltpu.VMEM((ZCH, HD), jnp.float32),
        pltpu.VMEM_SHARED((N_PAD, HD), jnp.float32),
        pltpu.SemaphoreType.DMA,
        pltpu.SemaphoreType.DMA,
    ],
)
def _seg3(src_hbm, dst_hbm, h00, h01, h10, h11, h20, h21, acc_out,
          srcv, dstv, bufs, zacc, accsh, semg, sems):
    c = lax.axis_index("c")
    s = lax.axis_index("s")
    wid = c * NS + s

    _zero_rows(zacc, ZCH, HD)
    base_n = s * NPS
    fast = c == jnp.int32(FAST_C)
    base_ch = jnp.where(fast, s * ECF, NS * ECF + s * ECS)
    nch = jnp.where(fast, jnp.int32(ECF), jnp.int32(ECS))
    nk = jnp.where(fast, jnp.int32(ECF // NBUF), jnp.int32(ECS // NBUF))
    lastc = nch - 1
    pltpu.sync_copy(src_hbm.at[pl.ds(base_ch * ECH, E_STAGE)], srcv)
    pltpu.sync_copy(dst_hbm.at[pl.ds(base_ch, ECF)], dstv)

    halves = ((h00, h01), (h10, h11), (h20, h21))
    for m in range(M):
        for half in range(2):
            for q in range(NPS // ZCH):
                pltpu.sync_copy(zacc, accsh.at[pl.ds(base_n + q * ZCH, ZCH)])
            plsc.subcore_barrier()

            _seg_pass(halves[m][half], srcv, dstv, bufs, accsh, semg, sems,
                      nk, lastc)
            plsc.subcore_barrier()
            pltpu.sync_copy(
                accsh.at[pl.ds(base_n, NPS)],
                acc_out.at[c, jnp.int32(m), jnp.int32(half), pl.ds(base_n, NPS)])
            plsc.subcore_barrier()


@functools.partial(
    pl.kernel,
    out_type=jax.ShapeDtypeStruct((NW, P_STAGE), jnp.float32),
    mesh=_mesh,
    compiler_params=pltpu.CompilerParams(needs_layout_passes=False),
    scratch_types=[
        pltpu.VMEM((P_STAGE,), jnp.int32),
        pltpu.VMEM((P_STAGE,), jnp.int32),
        [pltpu.VMEM((PCH, M * D), jnp.float32) for _ in range(2)],
        [pltpu.VMEM((PCH, M * D), jnp.float32) for _ in range(2)],
        pltpu.VMEM((P_STAGE,), jnp.float32),
        pltpu.SemaphoreType.DMA,
    ],
)
def _decode(z_hbm, ps_hbm, pd_hbm, out_hbm, ipsv, ipdv, zs, zd, outv, sem):
    c = lax.axis_index("c")
    s = lax.axis_index("s")
    wid = c * NS + s
    MD = M * D
    fast = c == jnp.int32(FAST_C)
    base_ch = jnp.where(fast, s * PCF, NS * PCF + s * PCS)
    nk2 = jnp.where(fast, jnp.int32(PCF // 2), jnp.int32(PCS // 2))
    lastc = jnp.where(fast, jnp.int32(PCF), jnp.int32(PCS)) - 1

    pltpu.sync_copy(ps_hbm.at[pl.ds(base_ch * PCH, P_STAGE)], ipsv)
    pltpu.sync_copy(pd_hbm.at[pl.ds(base_ch * PCH, P_STAGE)], ipdv)

    def fire(chunk, slot):
        off = chunk * PCH
        pltpu.async_copy(z_hbm.at[ipsv.at[pl.ds(off, PCH)]], zs[slot], sem)
        pltpu.async_copy(z_hbm.at[ipdv.at[pl.ds(off, PCH)]], zd[slot], sem)

    def drain(chunk, slot):
        off = chunk * PCH
        pltpu.make_async_copy(
            z_hbm.at[ipsv.at[pl.ds(off, PCH)]], zs[slot], sem).wait()
        pltpu.make_async_copy(
            z_hbm.at[ipdv.at[pl.ds(off, PCH)]], zd[slot], sem).wait()

    last_lane = lax.iota(jnp.int32, 16) == jnp.int32(15)

    def compute(chunk, slot):
        zsv, zdv = zs[slot], zd[slot]
        obase = chunk * PCH

        def pbody(p, _):
            a0 = zsv[p, pl.ds(0, 16)] * zdv[p, pl.ds(0, 16)]
            a1 = zsv[p, pl.ds(16, 16)] * zdv[p, pl.ds(16, 16)]
            a2 = zsv[p, pl.ds(32, 16)] * zdv[p, pl.ds(32, 16)]
            accs = [a0, a1, a2]
            for j in range(3, MD // 16):
                accs[j % 3] = accs[j % 3] + (zsv[p, pl.ds(j * 16, 16)]
                                             * zdv[p, pl.ds(j * 16, 16)])
            cum = plsc.cumsum((accs[0] + accs[1] + accs[2])
                              * jnp.float32(1.0 / M))
            idx = jnp.broadcast_to(obase + p, (16,)).astype(jnp.int32)
            plsc.store_scatter(outv, [idx], cum, mask=last_lane)
            return jnp.int32(0)

        lax.fori_loop(jnp.int32(0), jnp.int32(PCH), pbody, jnp.int32(0))

    fire(jnp.int32(0), 0)

    def body(k2, _):
        a = 2 * k2
        b = a + 1
        fire(b, 1)
        drain(a, 0)
        compute(a, 0)
        fire(jnp.minimum(a + 2, lastc), 0)
        drain(b, 1)
        compute(b, 1)
        return jnp.int32(0)

    lax.fori_loop(jnp.int32(0), nk2, body, jnp.int32(0))
    drain(lastc, 0)
    pltpu.sync_copy(outv, out_hbm.at[wid])


def _i0():
    return jnp.int32(0)

RB = 1000
NB = N // RB


def _combine_halves(acc4):
    return jnp.concatenate([acc4[0, 0] + acc4[1, 0], acc4[0, 1] + acc4[1, 1]],
                           axis=-1)


def _layer1_body(x_ref, acc_ref, deg_ref, ws_ref, wn_ref, b_ref, out_ref):
    deg = jnp.maximum(deg_ref[0, :, 0:1] + deg_ref[1, :, 0:1], jnp.float32(1.0))
    agg = _combine_halves(acc_ref[...]) / deg
    h = (jnp.dot(x_ref[...], ws_ref[0], preferred_element_type=jnp.float32)
         + jnp.dot(agg, wn_ref[0], preferred_element_type=jnp.float32)
         + b_ref[0, 0:1])
    h = jnp.maximum(h, jnp.float32(0.0))
    out_ref[0, 0] = h[:, :HD]
    out_ref[0, 1] = h[:, HD:]


def _layer1(x, accp, degp, ws1, wn1, b1):
    return pl.pallas_call(
        _layer1_body,
        grid=(M, NB),
        in_specs=[
            pl.BlockSpec((RB, D), lambda m, i: (i, _i0())),
            pl.BlockSpec((NC, 2, RB, HD), lambda m, i: (_i0(), _i0(), i, _i0())),
            pl.BlockSpec((NC, RB, 16), lambda m, i: (_i0(), i, _i0())),
            pl.BlockSpec((1, D, D), lambda m, i: (m, _i0(), _i0())),
            pl.BlockSpec((1, D, D), lambda m, i: (m, _i0(), _i0())),
            pl.BlockSpec((1, 8, D), lambda m, i: (m, _i0(), _i0())),
        ],
        out_specs=pl.BlockSpec((1, 2, RB, HD), lambda m, i: (m, _i0(), i, _i0())),
        out_shape=jax.ShapeDtypeStruct((M, 2, N, HD), jnp.float32),
    )(x, accp, degp, ws1, wn1, b1)


def _layer2_body(h_ref, acc_ref, deg_ref, ws_ref, wn_ref, b_ref, out_ref):
    deg = jnp.maximum(deg_ref[0, :, 0:1] + deg_ref[1, :, 0:1], jnp.float32(1.0))
    agg = _combine_halves(acc_ref[:, 0]) / deg
    h = jnp.concatenate([h_ref[0, 0], h_ref[0, 1]], axis=-1)
    z = (jnp.dot(h, ws_ref[0], preferred_element_type=jnp.float32)
         + jnp.dot(agg, wn_ref[0], preferred_element_type=jnp.float32)
         + b_ref[0, 0:1])
    out_ref[...] = z


def _layer2(h, accp2, degp, ws2, wn2, b2):
    return pl.pallas_call(
        _layer2_body,
        grid=(M, NB),
        in_specs=[
            pl.BlockSpec((1, 2, RB, HD), lambda m, i: (m, _i0(), i, _i0())),
            pl.BlockSpec((NC, 1, 2, RB, HD),
                         lambda m, i: (_i0(), m, _i0(), i, _i0())),
            pl.BlockSpec((NC, RB, 16), lambda m, i: (_i0(), i, _i0())),
            pl.BlockSpec((1, D, D), lambda m, i: (m, _i0(), _i0())),
            pl.BlockSpec((1, D, D), lambda m, i: (m, _i0(), _i0())),
            pl.BlockSpec((1, 8, D), lambda m, i: (m, _i0(), _i0())),
        ],
        out_specs=pl.BlockSpec((RB, D), lambda m, i: (i, m)),
        out_shape=jax.ShapeDtypeStruct((N, M * D), jnp.float32),
    )(h, accp2, degp, ws2, wn2, b2)


def kernel(x, edge_index, edge_pairs, Wself1, Wneigh1, b1, Wself2, Wneigh2, b2):
    x = x.astype(jnp.float32)
    src = edge_index[0].astype(jnp.int32)
    dst = edge_index[1].astype(jnp.int32)
    ps = edge_pairs[0].astype(jnp.int32)
    pd = edge_pairs[1].astype(jnp.int32)
    epad_src = jnp.zeros((E_PAD - E,), jnp.int32)
    epad_dst = jnp.full((E_PAD - E,), N_PAD - 1, jnp.int32)
    src_p = jnp.concatenate([src, epad_src])
    dst_p = jnp.concatenate([dst, epad_dst]).reshape(E_PAD // ECH, ECH)
    ppad = jnp.zeros((P_PAD - P,), jnp.int32)
    ps_p = jnp.concatenate([ps, ppad])
    pd_p = jnp.concatenate([pd, ppad])
    ws1 = Wself1.astype(jnp.float32)
    wn1 = Wneigh1.astype(jnp.float32)
    ws2 = Wself2.astype(jnp.float32)
    wn2 = Wneigh2.astype(jnp.float32)
    b1f = jnp.broadcast_to(b1.astype(jnp.float32)[:, None, :], (M, 8, D))
    b2f = jnp.broadcast_to(b2.astype(jnp.float32)[:, None, :], (M, 8, D))

    accp, degp = _seg1(src_p, dst_p, x[:, :HD], x[:, HD:])
    h = _layer1(x, accp, degp, ws1, wn1, b1f)
    accp2 = _seg3(src_p, dst_p, h[0, 0], h[0, 1], h[1, 0], h[1, 1],
                  h[2, 0], h[2, 1])
    z = _layer2(h, accp2, degp, ws2, wn2, b2f)
    lbuf = _decode(z, ps_p, pd_p)
    fast_rows = lbuf[FAST_C * NS:(FAST_C + 1) * NS].reshape(-1)
    slow_c = 1 - FAST_C
    slow_rows = lbuf[slow_c * NS:(slow_c + 1) * NS, :PCS * PCH].reshape(-1)
    logits = jnp.concatenate([fast_rows, slow_rows])
    return logits[:P].astype(jnp.float64)


# speedup vs baseline: 1.0183x; 1.0183x over previous
import functools

import jax
import jax.numpy as jnp
from jax import lax
from jax.experimental import pallas as pl
from jax.experimental.pallas import tpu as pltpu
from jax.experimental.pallas import tpu_sc as plsc

N = 10000
D = 128
HD = D // 2
M = 3
E = 320000
P = 200000

NC = 2
NS = 16
NW = NC * NS

ECH = 128
NBUF = 4
FAST_C = 0
ECF = 132
ECS = 28
E_CHUNKS = NS * (ECF + ECS)
E_STAGE = ECF * ECH
E_PAD = (NS * ECF + (NS - 1) * ECS + ECF) * ECH

PCH = 56
PCF = 166
PCS = 58
P_CHUNKS = NS * (PCF + PCS)
P_STAGE = PCF * PCH
P_PAD = (NS * PCF + (NS - 1) * PCS + PCF) * PCH

N_PAD = 10008
NPS = N // NS
ZCH = 125

_mesh = plsc.VectorSubcoreMesh(core_axis_name="c", subcore_axis_name="s")


def _zero_rows(ref, nrows, width):
    def body(i, _):
        for j in range(width // 16):
            ref[i, pl.ds(j * 16, 16)] = jnp.zeros((16,), jnp.float32)
        return jnp.int32(0)
    lax.fori_loop(jnp.int32(0), jnp.int32(nrows), body, jnp.int32(0))


def _ones_rows(ref, nrows):
    def body(i, _):
        ref[i, :] = jnp.ones((16,), jnp.float32)
        return jnp.int32(0)
    lax.fori_loop(jnp.int32(0), jnp.int32(nrows), body, jnp.int32(0))


def _seg_pass(feat_hbm, srcv, dstv, bufs, accsh, semg, sems, nk, lastc,
              deg=None):
    def g_src(chunk):
        return feat_hbm.at[srcv.at[pl.ds(chunk * ECH, ECH)]]

    for j in range(NBUF):
        pltpu.async_copy(g_src(jnp.int32(j)), bufs[j], semg)

    def body(k, _):
        for j in range(NBUF):
            i = k * NBUF + j
            pltpu.make_async_copy(g_src(i), bufs[j], semg).wait()
            pltpu.async_copy(bufs[j], accsh.at[dstv.at[i]], sems, add=True)
            if deg is not None:
                onesv, degsh, semd = deg
                pltpu.async_copy(onesv, degsh.at[dstv.at[i]], semd, add=True)
                pltpu.make_async_copy(onesv, degsh.at[dstv.at[i]], semd).wait()
            pltpu.make_async_copy(bufs[j], accsh.at[dstv.at[i]], sems).wait()
            nxt = jnp.minimum(i + NBUF, lastc)
            pltpu.async_copy(g_src(nxt), bufs[j], semg)
        return jnp.int32(0)

    lax.fori_loop(jnp.int32(0), nk, body, jnp.int32(0))
    for j in range(NBUF):
        pltpu.make_async_copy(g_src(lastc), bufs[j], semg).wait()


@functools.partial(
    pl.kernel,
    out_type=(
        jax.ShapeDtypeStruct((NC, 2, N_PAD, HD), jnp.float32),
        jax.ShapeDtypeStruct((NC, N_PAD, 16), jnp.float32),
    ),
    mesh=_mesh,
    compiler_params=pltpu.CompilerParams(use_tc_tiling_on_sc=False),
    scratch_types=[
        pltpu.VMEM((E_STAGE,), jnp.int32),
        pltpu.VMEM((ECF, ECH), jnp.int32),
        [pltpu.VMEM((ECH, HD), jnp.float32) for _ in range(NBUF)],
        pltpu.VMEM((ECH, 16), jnp.float32),
        pltpu.VMEM((ZCH, HD), jnp.float32),
        pltpu.VMEM((ZCH, 16), jnp.float32),
        pltpu.VMEM_SHARED((N_PAD, HD), jnp.float32),
        pltpu.VMEM_SHARED((N_PAD, 16), jnp.float32),
        pltpu.SemaphoreType.DMA,
        pltpu.SemaphoreType.DMA,
        pltpu.SemaphoreType.DMA,
    ],
)
def _seg1(src_hbm, dst_hbm, x0_hbm, x1_hbm, acc_out, deg_out,
          srcv, dstv, bufs, onesv, zacc, zdeg, accsh, degsh,
          semg, sems, semd):
    c = lax.axis_index("c")
    s = lax.axis_index("s")
    wid = c * NS + s

    _zero_rows(zacc, ZCH, HD)
    _zero_rows(zdeg, ZCH, 16)
    _ones_rows(onesv, ECH)

    base_n = s * NPS
    fast = c == jnp.int32(FAST_C)
    base_ch = jnp.where(fast, s * ECF, NS * ECF + s * ECS)
    nch = jnp.where(fast, jnp.int32(ECF), jnp.int32(ECS))
    nk = jnp.where(fast, jnp.int32(ECF // NBUF), jnp.int32(ECS // NBUF))
    lastc = nch - 1
    pltpu.sync_copy(src_hbm.at[pl.ds(base_ch * ECH, E_STAGE)], srcv)
    pltpu.sync_copy(dst_hbm.at[pl.ds(base_ch, ECF)], dstv)

    for half, xh_hbm in enumerate((x0_hbm, x1_hbm)):
        for q in range(NPS // ZCH):
            pltpu.sync_copy(zacc, accsh.at[pl.ds(base_n + q * ZCH, ZCH)])
            if half == 0:
                pltpu.sync_copy(zdeg, degsh.at[pl.ds(base_n + q * ZCH, ZCH)])
        plsc.subcore_barrier()

        _seg_pass(xh_hbm, srcv, dstv, bufs, accsh, semg, sems, nk, lastc,
                  deg=(onesv, degsh, semd) if half == 0 else None)
        plsc.subcore_barrier()

        pltpu.sync_copy(accsh.at[pl.ds(base_n, NPS)],
                        acc_out.at[c, jnp.int32(half), pl.ds(base_n, NPS)])
        if half == 0:
            pltpu.sync_copy(degsh.at[pl.ds(base_n, NPS)],
                            deg_out.at[c, pl.ds(base_n, NPS)])
        plsc.subcore_barrier()


@functools.partial(
    pl.kernel,
    out_type=jax.ShapeDtypeStruct((NC, M, 2, N_PAD, HD), jnp.float32),
    mesh=_mesh,
    compiler_params=pltpu.CompilerParams(use_tc_tiling_on_sc=False),
    scratch_types=[
        pltpu.VMEM((E_STAGE,), jnp.int32),
        pltpu.VMEM((ECF, ECH), jnp.int32),
        [pltpu.VMEM((ECH, HD), jnp.float32) for _ in range(NBUF)],
        pltpu.VMEM((ZCH, HD), jnp.float32),
        pltpu.VMEM_SHARED((N_PAD, HD), jnp.float32),
        pltpu.SemaphoreType.DMA,
        pltpu.SemaphoreType.DMA,
    ],
)
def _seg3(src_hbm, dst_hbm, h00, h01, h10, h11, h20, h21, acc_out,
          srcv, dstv, bufs, zacc, accsh, semg, sems):
    c = lax.axis_index("c")
    s = lax.axis_index("s")
    wid = c * NS + s

    _zero_rows(zacc, ZCH, HD)
    base_n = s * NPS
    fast = c == jnp.int32(FAST_C)
    base_ch = jnp.where(fast, s * ECF, NS * ECF + s * ECS)
    nch = jnp.where(fast, jnp.int32(ECF), jnp.int32(ECS))
    nk = jnp.where(fast, jnp.int32(ECF // NBUF), jnp.int32(ECS // NBUF))
    lastc = nch - 1
    pltpu.sync_copy(src_hbm.at[pl.ds(base_ch * ECH, E_STAGE)], srcv)
    pltpu.sync_copy(dst_hbm.at[pl.ds(base_ch, ECF)], dstv)

    halves = ((h00, h01), (h10, h11), (h20, h21))
    for m in range(M):
        for half in range(2):
            for q in range(NPS // ZCH):
                pltpu.sync_copy(zacc, accsh.at[pl.ds(base_n + q * ZCH, ZCH)])
            plsc.subcore_barrier()

            _seg_pass(halves[m][half], srcv, dstv, bufs, accsh, semg, sems,
                      nk, lastc)
            plsc.subcore_barrier()
            pltpu.sync_copy(
                accsh.at[pl.ds(base_n, NPS)],
                acc_out.at[c, jnp.int32(m), jnp.int32(half), pl.ds(base_n, NPS)])
            plsc.subcore_barrier()


@functools.partial(
    pl.kernel,
    out_type=jax.ShapeDtypeStruct((NW, P_STAGE), jnp.float32),
    mesh=_mesh,
    compiler_params=pltpu.CompilerParams(needs_layout_passes=False),
    scratch_types=[
        pltpu.VMEM((P_STAGE,), jnp.int32),
        pltpu.VMEM((P_STAGE,), jnp.int32),
        [pltpu.VMEM((PCH, M * D), jnp.float32) for _ in range(2)],
        [pltpu.VMEM((PCH, M * D), jnp.float32) for _ in range(2)],
        pltpu.VMEM((P_STAGE,), jnp.float32),
        pltpu.SemaphoreType.DMA,
    ],
)
def _decode(z_hbm, ps_hbm, pd_hbm, out_hbm, ipsv, ipdv, zs, zd, outv, sem):
    c = lax.axis_index("c")
    s = lax.axis_index("s")
    wid = c * NS + s
    MD = M * D
    fast = c == jnp.int32(FAST_C)
    base_ch = jnp.where(fast, s * PCF, NS * PCF + s * PCS)
    nk2 = jnp.where(fast, jnp.int32(PCF // 2), jnp.int32(PCS // 2))
    lastc = jnp.where(fast, jnp.int32(PCF), jnp.int32(PCS)) - 1

    pltpu.sync_copy(ps_hbm.at[pl.ds(base_ch * PCH, P_STAGE)], ipsv)
    pltpu.sync_copy(pd_hbm.at[pl.ds(base_ch * PCH, P_STAGE)], ipdv)

    def fire(chunk, slot):
        off = chunk * PCH
        pltpu.async_copy(z_hbm.at[ipsv.at[pl.ds(off, PCH)]], zs[slot], sem)
        pltpu.async_copy(z_hbm.at[ipdv.at[pl.ds(off, PCH)]], zd[slot], sem)

    def drain(chunk, slot):
        off = chunk * PCH
        pltpu.make_async_copy(
            z_hbm.at[ipsv.at[pl.ds(off, PCH)]], zs[slot], sem).wait()
        pltpu.make_async_copy(
            z_hbm.at[ipdv.at[pl.ds(off, PCH)]], zd[slot], sem).wait()

    last_lane = lax.iota(jnp.int32, 16) == jnp.int32(15)

    def compute(chunk, slot):
        zsv, zdv = zs[slot], zd[slot]
        obase = chunk * PCH

        def pbody(p, _):
            a0 = zsv[p, pl.ds(0, 16)] * zdv[p, pl.ds(0, 16)]
            a1 = zsv[p, pl.ds(16, 16)] * zdv[p, pl.ds(16, 16)]
            a2 = zsv[p, pl.ds(32, 16)] * zdv[p, pl.ds(32, 16)]
            accs = [a0, a1, a2]
            for j in range(3, MD // 16):
                accs[j % 3] = accs[j % 3] + (zsv[p, pl.ds(j * 16, 16)]
                                             * zdv[p, pl.ds(j * 16, 16)])
            cum = plsc.cumsum((accs[0] + accs[1] + accs[2])
                              * jnp.float32(1.0 / M))
            idx = jnp.broadcast_to(obase + p, (16,)).astype(jnp.int32)
            plsc.store_scatter(outv, [idx], cum, mask=last_lane)
            return jnp.int32(0)

        lax.fori_loop(jnp.int32(0), jnp.int32(PCH), pbody, jnp.int32(0))

    fire(jnp.int32(0), 0)

    def body(k2, _):
        a = 2 * k2
        b = a + 1
        fire(b, 1)
        drain(a, 0)
        compute(a, 0)
        fire(jnp.minimum(a + 2, lastc), 0)
        drain(b, 1)
        compute(b, 1)
        return jnp.int32(0)

    lax.fori_loop(jnp.int32(0), nk2, body, jnp.int32(0))
    drain(lastc, 0)
    pltpu.sync_copy(outv, out_hbm.at[wid])


def _i0():
    return jnp.int32(0)

RB = 1000
NB = N // RB


def _combine_halves(acc4):
    return jnp.concatenate([acc4[0, 0] + acc4[1, 0], acc4[0, 1] + acc4[1, 1]],
                           axis=-1)


def _layer1_body(x_ref, acc_ref, deg_ref, ws_ref, wn_ref, b_ref, out_ref):
    deg = jnp.maximum(deg_ref[0, :, 0:1] + deg_ref[1, :, 0:1], jnp.float32(1.0))
    agg = _combine_halves(acc_ref[...]) / deg
    h = (jnp.dot(x_ref[...], ws_ref[0], preferred_element_type=jnp.float32)
         + jnp.dot(agg, wn_ref[0], preferred_element_type=jnp.float32)
         + b_ref[0, 0:1])
    h = jnp.maximum(h, jnp.float32(0.0))
    out_ref[0, 0] = h[:, :HD]
    out_ref[0, 1] = h[:, HD:]


def _layer1(x, accp, degp, ws1, wn1, b1):
    return pl.pallas_call(
        _layer1_body,
        grid=(M, NB),
        in_specs=[
            pl.BlockSpec((RB, D), lambda m, i: (i, _i0())),
            pl.BlockSpec((NC, 2, RB, HD), lambda m, i: (_i0(), _i0(), i, _i0())),
            pl.BlockSpec((NC, RB, 16), lambda m, i: (_i0(), i, _i0())),
            pl.BlockSpec((1, D, D), lambda m, i: (m, _i0(), _i0())),
            pl.BlockSpec((1, D, D), lambda m, i: (m, _i0(), _i0())),
            pl.BlockSpec((1, 8, D), lambda m, i: (m, _i0(), _i0())),
        ],
        out_specs=pl.BlockSpec((1, 2, RB, HD), lambda m, i: (m, _i0(), i, _i0())),
        out_shape=jax.ShapeDtypeStruct((M, 2, N, HD), jnp.float32),
    )(x, accp, degp, ws1, wn1, b1)


def _layer2_body(h_ref, acc_ref, deg_ref, ws_ref, wn_ref, b_ref, out_ref):
    deg = jnp.maximum(deg_ref[0, :, 0:1] + deg_ref[1, :, 0:1], jnp.float32(1.0))
    agg = _combine_halves(acc_ref[:, 0]) / deg
    h = jnp.concatenate([h_ref[0, 0], h_ref[0, 1]], axis=-1)
    z = (jnp.dot(h, ws_ref[0], preferred_element_type=jnp.float32)
         + jnp.dot(agg, wn_ref[0], preferred_element_type=jnp.float32)
         + b_ref[0, 0:1])
    out_ref[...] = z


def _layer2(h, accp2, degp, ws2, wn2, b2):
    return pl.pallas_call(
        _layer2_body,
        grid=(M, NB),
        in_specs=[
            pl.BlockSpec((1, 2, RB, HD), lambda m, i: (m, _i0(), i, _i0())),
            pl.BlockSpec((NC, 1, 2, RB, HD),
                         lambda m, i: (_i0(), m, _i0(), i, _i0())),
            pl.BlockSpec((NC, RB, 16), lambda m, i: (_i0(), i, _i0())),
            pl.BlockSpec((1, D, D), lambda m, i: (m, _i0(), _i0())),
            pl.BlockSpec((1, D, D), lambda m, i: (m, _i0(), _i0())),
            pl.BlockSpec((1, 8, D), lambda m, i: (m, _i0(), _i0())),
        ],
        out_specs=pl.BlockSpec((RB, D), lambda m, i: (i, m)),
        out_shape=jax.ShapeDtypeStruct((N, M * D), jnp.float32),
    )(h, accp2, degp, ws2, wn2, b2)


def kernel(x, edge_index, edge_pairs, Wself1, Wneigh1, b1, Wself2, Wneigh2, b2):
    x = x.astype(jnp.float32)
    src = edge_index[0].astype(jnp.int32)
    dst = edge_index[1].astype(jnp.int32)
    ps = edge_pairs[0].astype(jnp.int32)
    pd = edge_pairs[1].astype(jnp.int32)
    epad_src = jnp.zeros((E_PAD - E,), jnp.int32)
    epad_dst = jnp.full((E_PAD - E,), N_PAD - 1, jnp.int32)
    src_p = jnp.concatenate([src, epad_src])
    dst_p = jnp.concatenate([dst, epad_dst]).reshape(E_PAD // ECH, ECH)
    ppad = jnp.zeros((P_PAD - P,), jnp.int32)
    ps_p = jnp.concatenate([ps, ppad])
    pd_p = jnp.concatenate([pd, ppad])
    ws1 = Wself1.astype(jnp.float32)
    wn1 = Wneigh1.astype(jnp.float32)
    ws2 = Wself2.astype(jnp.float32)
    wn2 = Wneigh2.astype(jnp.float32)
    b1f = jnp.broadcast_to(b1.astype(jnp.float32)[:, None, :], (M, 8, D))
    b2f = jnp.broadcast_to(b2.astype(jnp.float32)[:, None, :], (M, 8, D))

    accp, degp = _seg1(src_p, dst_p, x[:, :HD], x[:, HD:])
    h = _layer1(x, accp, degp, ws1, wn1, b1f)
    accp2 = _seg3(src_p, dst_p, h[0, 0], h[0, 1], h[1, 0], h[1, 1],
                  h[2, 0], h[2, 1])
    z = _layer2(h, accp2, degp, ws2, wn2, b2f)
    lbuf = _decode(z, ps_p, pd_p)
    fast_rows = lbuf[FAST_C * NS:(FAST_C + 1) * NS].reshape(-1)
    slow_c = 1 - FAST_C
    slow_rows = lbuf[slow_c * NS:(slow_c + 1) * NS, :PCS * PCH].reshape(-1)
    logits = jnp.concatenate([fast_rows, slow_rows])
    return logits[:P].astype(jnp.float64)
